# baseline (speedup 1.0000x reference)
import functools

import jax
import jax.numpy as jnp
from jax import lax
from jax.experimental import pallas as pl
from jax.experimental.pallas import tpu as pltpu
from jax.experimental.pallas import tpu_sc as plsc

P_DIM = 128
E = 320000
NC, NS, L = 2, 16, 16
NW = NC * NS
RPW = E // NW
CH = 400
NCH = RPW // CH
CHW = CH * P_DIM


def _sc_body(mu_hbm, aux_hbm, mu_out, aux_out, buf, accv, auxv, auxbuf,
             sem0, sem1):
    wid = lax.axis_index("s") * NC + lax.axis_index("c")
    base = wid * (RPW * P_DIM)
    sems = (sem0, sem1)

    def start(c):
        return pltpu.async_copy(
            mu_hbm.at[pl.ds(base + c * CHW, CHW)],
            buf.at[pl.ds((c % 2) * CHW, CHW)],
            sems[c % 2])

    pend = start(0)

    for a in range(4):
        pltpu.sync_copy(aux_hbm.at[pl.ds(a * E + wid * RPW, RPW)], auxbuf)

        def aux_body(i, pn):
            v = auxbuf[pl.ds(i * L, L)]
            return (pn[0] + jnp.maximum(v, 0.0), pn[1] + jnp.maximum(-v, 0.0))

        z = jnp.zeros((L,), jnp.float32)
        p, n = lax.fori_loop(0, RPW // L, aux_body, (z, z))
        auxv[pl.ds((2 * a) * L, L)] = p
        auxv[pl.ds((2 * a + 1) * L, L)] = n
    pltpu.sync_copy(auxv, aux_out.at[wid])

    accs = tuple(jnp.zeros((L,), jnp.float32) for _ in range(8))
    for c in range(NCH):
        nxt = start(c + 1) if c + 1 < NCH else None
        pend.wait()
        pend = nxt
        b0 = (c % 2) * CHW

        def row_body(i, acc, b0=b0):
            off = b0 + i * (2 * P_DIM)
            a = list(acc)
            for u in range(2):
                for k in range(8):
                    a[k] = a[k] + buf[pl.ds(off + u * P_DIM + k * L, L)]
            return tuple(a)

        accs = lax.fori_loop(0, CH // 2, row_body, accs)
    for k in range(8):
        accv[pl.ds(k * L, L)] = accs[k]
    pltpu.sync_copy(accv, mu_out.at[wid])


_sc_call = pl.kernel(
    _sc_body,
    out_type=[jax.ShapeDtypeStruct((NW, P_DIM), jnp.float32),
              jax.ShapeDtypeStruct((NW, P_DIM), jnp.float32)],
    mesh=plsc.VectorSubcoreMesh(core_axis_name="c", subcore_axis_name="s",
                                num_cores=NC, num_subcores=NS),
    scratch_types=[
        pltpu.VMEM((2 * CHW,), jnp.float32),
        pltpu.VMEM((P_DIM,), jnp.float32),
        pltpu.VMEM((P_DIM,), jnp.float32),
        pltpu.VMEM((RPW,), jnp.float32),
        pltpu.SemaphoreType.DMA,
        pltpu.SemaphoreType.DMA,
    ],
)


def _combine_body(pm_ref, pa_ref, w1t_ref, w2t_ref, w4t_ref, w6t_ref,
                  w8t_ref, wv_ref, w10t_ref, xi_ref, out_ref):
    s = jnp.sum(pm_ref[...], axis=0, keepdims=True)
    pa = jnp.sum(pa_ref[...], axis=0, keepdims=True)
    seg = lax.broadcasted_iota(jnp.int32, (1, P_DIM), 1) // L
    wv = wv_ref[...]
    tmp = jnp.dot(s, w1t_ref[...], preferred_element_type=jnp.float32)
    wts = (w2t_ref, w4t_ref, w6t_ref, w8t_ref)
    for a in range(4):
        p = jnp.sum(jnp.where(seg == 2 * a, pa, 0.0))
        n = jnp.sum(jnp.where(seg == 2 * a + 1, pa, 0.0))
        v = (p * jnp.maximum(wv[a:a + 1], 0.0)
             + n * jnp.maximum(-wv[a:a + 1], 0.0))
        tmp += jnp.dot(v, wts[a][...], preferred_element_type=jnp.float32)
    tmp += jnp.dot(xi_ref[...], w10t_ref[...],
                   preferred_element_type=jnp.float32)
    out_ref[...] = jnp.maximum(tmp, 0.0)


def kernel(xi, mu_N, h, hc, s, sc, W1, W2, W3, W4, W5, W6, W7, W8, W9, W10):
    mu_flat = mu_N.reshape(-1)
    aux_flat = jnp.concatenate([h[:, 0], hc[:, 0], s[:, 0], sc[:, 0]])
    wv = jnp.stack([W3[:, 0], W5[:, 0], W7[:, 0], W9[:, 0]])
    part_mu, part_aux = _sc_call(mu_flat, aux_flat)

    full = lambda shape: pl.BlockSpec(shape, lambda: (0,) * len(shape))
    out = pl.pallas_call(
        _combine_body,
        in_specs=[
            full((NW, P_DIM)), full((NW, P_DIM)),
            full((P_DIM, P_DIM)), full((P_DIM, P_DIM)), full((P_DIM, P_DIM)),
            full((P_DIM, P_DIM)), full((P_DIM, P_DIM)),
            full((4, P_DIM)),
            full((2, P_DIM)),
            full((1, 2)),
        ],
        out_specs=full((1, P_DIM)),
        out_shape=jax.ShapeDtypeStruct((1, P_DIM), jnp.float32),
    )(part_mu, part_aux, W1.T, W2.T, W4.T, W6.T, W8.T, wv, W10.T,
      xi.reshape(1, 2))
    return out.reshape(P_DIM)


# speedup vs baseline: 1.6418x; 1.6418x over previous
import jax
import jax.numpy as jnp
from jax import lax
from jax.experimental import pallas as pl
from jax.experimental.pallas import tpu as pltpu
from jax.experimental.pallas import tpu_sc as plsc

P_DIM = 128
E = 320000
NC, NS, L = 2, 16, 16
NW = NC * NS
RPW = E // NW
CH = 400
NCH = RPW // CH
CHW = CH * P_DIM
RU = 4
AUX_R = E // P_DIM


def _sc_body(mu_hbm, mu_out, buf, accv, sem0, sem1):
    wid = lax.axis_index("s") * NC + lax.axis_index("c")
    base = wid * (RPW * P_DIM)
    sems = (sem0, sem1)

    def start(c):
        return pltpu.async_copy(
            mu_hbm.at[pl.ds(base + c * CHW, CHW)],
            buf.at[pl.ds((c % 2) * CHW, CHW)],
            sems[c % 2])

    pend = start(0)
    accs = tuple(jnp.zeros((L,), jnp.float32) for _ in range(8))
    for c in range(NCH):
        nxt = start(c + 1) if c + 1 < NCH else None
        pend.wait()
        pend = nxt
        b0 = (c % 2) * CHW

        def row_body(i, acc, b0=b0):
            off = b0 + i * (RU * P_DIM)
            a = list(acc)
            for u in range(RU):
                for k in range(8):
                    a[k] = a[k] + buf[pl.ds(off + u * P_DIM + k * L, L)]
            return tuple(a)

        accs = lax.fori_loop(0, CH // RU, row_body, accs)
    for k in range(8):
        accv[pl.ds(k * L, L)] = accs[k]
    pltpu.sync_copy(accv, mu_out.at[wid])


_sc_call = pl.kernel(
    _sc_body,
    out_type=jax.ShapeDtypeStruct((NW, P_DIM), jnp.float32),
    mesh=plsc.VectorSubcoreMesh(core_axis_name="c", subcore_axis_name="s",
                                num_cores=NC, num_subcores=NS),
    scratch_types=[
        pltpu.VMEM((2 * CHW,), jnp.float32),
        pltpu.VMEM((P_DIM,), jnp.float32),
        pltpu.SemaphoreType.DMA,
        pltpu.SemaphoreType.DMA,
    ],
)


def _combine_body(pm_ref, aux_ref, w1t_ref, w2t_ref, w4t_ref, w6t_ref,
                  w8t_ref, wv_ref, w10t_ref, xi_ref, out_ref):
    s = jnp.sum(pm_ref[...], axis=0, keepdims=True)
    aux = aux_ref[...]
    p = jnp.sum(jnp.maximum(aux, 0.0), axis=(1, 2), keepdims=True)[:, :, 0]
    n = jnp.sum(jnp.maximum(-aux, 0.0), axis=(1, 2), keepdims=True)[:, :, 0]
    wv = wv_ref[...]
    v = p * jnp.maximum(wv, 0.0) + n * jnp.maximum(-wv, 0.0)
    tmp = jnp.dot(s, w1t_ref[...], preferred_element_type=jnp.float32)
    tmp += jnp.dot(v[0:1], w2t_ref[...], preferred_element_type=jnp.float32)
    tmp += jnp.dot(v[1:2], w4t_ref[...], preferred_element_type=jnp.float32)
    tmp += jnp.dot(v[2:3], w6t_ref[...], preferred_element_type=jnp.float32)
    tmp += jnp.dot(v[3:4], w8t_ref[...], preferred_element_type=jnp.float32)
    tmp += jnp.dot(xi_ref[...], w10t_ref[...],
                   preferred_element_type=jnp.float32)
    out_ref[...] = jnp.maximum(tmp, 0.0)


def kernel(xi, mu_N, h, hc, s, sc, W1, W2, W3, W4, W5, W6, W7, W8, W9, W10):
    part_mu = _sc_call(mu_N.reshape(-1))
    aux = jnp.stack([h.reshape(AUX_R, P_DIM), hc.reshape(AUX_R, P_DIM),
                     s.reshape(AUX_R, P_DIM), sc.reshape(AUX_R, P_DIM)])
    wv = jnp.stack([W3[:, 0], W5[:, 0], W7[:, 0], W9[:, 0]])

    full = lambda shape: pl.BlockSpec(shape, lambda: (0,) * len(shape))
    out = pl.pallas_call(
        _combine_body,
        in_specs=[
            full((NW, P_DIM)),
            full((4, AUX_R, P_DIM)),
            full((P_DIM, P_DIM)), full((P_DIM, P_DIM)), full((P_DIM, P_DIM)),
            full((P_DIM, P_DIM)), full((P_DIM, P_DIM)),
            full((4, P_DIM)),
            full((2, P_DIM)),
            full((1, 2)),
        ],
        out_specs=full((1, P_DIM)),
        out_shape=jax.ShapeDtypeStruct((1, P_DIM), jnp.float32),
    )(part_mu, aux, W1.T, W2.T, W4.T, W6.T, W8.T, wv, W10.T,
      xi.reshape(1, 2))
    return out.reshape(P_DIM)
